# baseline (speedup 1.0000x reference)
import math

import jax
import jax.numpy as jnp
from jax.experimental import pallas as pl
from jax.experimental.pallas import tpu as pltpu

_N = 2048
_B = 2
_CD = 2
_TILE = 256
_NT = _N // _TILE
_GRID = _B * _NT
_NSTEPS = 8
_LOSS_DB = 0.1


def _encoder_kernel(adj_ref, encw_ref, swt_ref, iso_ref):
    logits_t = jax.lax.dot_general(
        encw_ref[...], adj_ref[0], (((1,), (1,)), ((), ())),
        preferred_element_type=jnp.float32)
    maskf = (jax.nn.sigmoid(logits_t) > 0.5).astype(jnp.float32)
    deg = jnp.sum(maskf, axis=0, keepdims=True)
    s = jnp.where(deg > 0, 1.0 / jnp.sqrt(jnp.maximum(deg, 1.0)), 0.0)
    swt_ref[0] = (maskf * s).astype(jnp.bfloat16)
    iso_ref[0] = (deg == 0.0).astype(jnp.float32)


def _walk_kernel(swt_ref, iso_ref, c4_ref, w1e_ref, w1o_ref, b1_ref,
                 w2_ref, b2_ref, out_ref):
    c4 = c4_ref[...]
    for b in range(_B):
        swt = swt_ref[b]
        iso = iso_ref[b]
        walker0 = jnp.full((4, _N), 1.0 / math.sqrt(_N * _CD),
                           dtype=jnp.float32)

        def _step(step, walker):
            ev = (c4[:, 0:1] * walker[0:1, :]
                  + c4[:, 1:2] * walker[1:2, :]
                  + c4[:, 2:3] * walker[2:3, :]
                  + c4[:, 3:4] * walker[3:4, :])
            contrib = jax.lax.dot_general(
                ev.astype(jnp.bfloat16), swt, (((1,), (1,)), ((), ())),
                preferred_element_type=jnp.float32)
            walker = contrib + iso * ev
            walker = walker * jnp.exp(step.astype(jnp.float32)
                                      * (-_LOSS_DB / 10.0))
            norm = jnp.sqrt(jnp.sum(walker * walker))
            return walker / (norm + 1e-08)

        walker = jax.lax.fori_loop(0, _NSTEPS, _step, walker0)
        p0 = walker[0:1, :] ** 2 + walker[1:2, :] ** 2
        p1 = walker[2:3, :] ** 2 + walker[3:4, :] ** 2
        h = jnp.maximum(
            jax.lax.dot_general(p0, w1e_ref[...], (((1,), (1,)), ((), ())),
                                preferred_element_type=jnp.float32)
            + jax.lax.dot_general(p1, w1o_ref[...], (((1,), (1,)), ((), ())),
                                  preferred_element_type=jnp.float32)
            + b1_ref[...], 0.0)
        out_b = jax.lax.dot_general(
            h, w2_ref[...], (((1,), (0,)), ((), ())),
            preferred_element_type=jnp.float32) + b2_ref[...]
        out_ref[pl.ds(b, 1), :] = out_b


def kernel(graph_adjacency, coin_operator, enc_W, enc_b, fe_W1, fe_b1,
           fe_W2, fe_b2):
    coin_c = coin_operator[..., 0] + 1j * coin_operator[..., 1]
    coin_c = coin_c / jnp.linalg.norm(coin_c)
    cr = jnp.real(coin_c).astype(jnp.float32)
    ci = jnp.imag(coin_c).astype(jnp.float32)
    c4 = jnp.stack([
        jnp.stack([cr[0, 0], -ci[0, 0], cr[0, 1], -ci[0, 1]]),
        jnp.stack([ci[0, 0], cr[0, 0], ci[0, 1], cr[0, 1]]),
        jnp.stack([cr[1, 0], -ci[1, 0], cr[1, 1], -ci[1, 1]]),
        jnp.stack([ci[1, 0], cr[1, 0], ci[1, 1], cr[1, 1]]),
    ])
    w1e = fe_W1[:, 0::2]
    w1o = fe_W1[:, 1::2]
    w2 = fe_W2.T
    b1 = fe_b1.reshape(1, 128)
    b2 = fe_b2.reshape(1, 64)

    swt, iso = pl.pallas_call(
        _encoder_kernel,
        grid=(_GRID,),
        in_specs=[
            pl.BlockSpec((1, _TILE, _N), lambda t: (t // _NT, t % _NT, 0)),
            pl.BlockSpec((_N, _N), lambda t: (0, 0)),
        ],
        out_specs=[
            pl.BlockSpec((1, _N, _TILE), lambda t: (t // _NT, 0, t % _NT)),
            pl.BlockSpec((1, 1, _TILE), lambda t: (t // _NT, 0, t % _NT)),
        ],
        out_shape=[
            jax.ShapeDtypeStruct((_B, _N, _N), jnp.bfloat16),
            jax.ShapeDtypeStruct((_B, 1, _N), jnp.float32),
        ],
    )(graph_adjacency, enc_W)

    out = pl.pallas_call(
        _walk_kernel,
        in_specs=[
            pl.BlockSpec((_B, _N, _N), lambda: (0, 0, 0)),
            pl.BlockSpec((_B, 1, _N), lambda: (0, 0, 0)),
            pl.BlockSpec((4, 4), lambda: (0, 0)),
            pl.BlockSpec((128, _N), lambda: (0, 0)),
            pl.BlockSpec((128, _N), lambda: (0, 0)),
            pl.BlockSpec((1, 128), lambda: (0, 0)),
            pl.BlockSpec((128, 64), lambda: (0, 0)),
            pl.BlockSpec((1, 64), lambda: (0, 0)),
        ],
        out_specs=pl.BlockSpec((_B, 64), lambda: (0, 0)),
        out_shape=jax.ShapeDtypeStruct((_B, 64), jnp.float32),
    )(swt, iso, c4, w1e, w1o, b1, w2, b2)
    return out


# speedup vs baseline: 1.6004x; 1.6004x over previous
import math

import jax
import jax.numpy as jnp
from jax.experimental import pallas as pl
from jax.experimental.pallas import tpu as pltpu

_N = 2048
_B = 2
_CD = 2
_TILE = 256
_NT = _N // _TILE
_GRID = _B * _NT
_NSTEPS = 8
_LOSS_DB = 0.1


def _encoder_kernel(adj_ref, encw_ref, swt_ref, iso_ref):
    logits_t = jax.lax.dot_general(
        encw_ref[...], adj_ref[0], (((1,), (1,)), ((), ())),
        preferred_element_type=jnp.float32)
    maskf = (jax.nn.sigmoid(logits_t) > 0.5).astype(jnp.float32)
    deg = jnp.sum(maskf, axis=0, keepdims=True)
    s = jnp.where(deg > 0, 1.0 / jnp.sqrt(jnp.maximum(deg, 1.0)), 0.0)
    swt_ref[0] = (maskf * s).astype(jnp.bfloat16)
    iso_ref[0] = (deg == 0.0).astype(jnp.float32)


def _walk_kernel(swt_ref, iso_ref, c4_ref, out_ref):
    c4 = c4_ref[...]
    for b in range(_B):
        swt = swt_ref[b]
        iso = iso_ref[b]
        walker0 = jnp.full((4, _N), 1.0 / math.sqrt(_N * _CD),
                           dtype=jnp.float32)

        def _step(step, walker):
            ev = (c4[:, 0:1] * walker[0:1, :]
                  + c4[:, 1:2] * walker[1:2, :]
                  + c4[:, 2:3] * walker[2:3, :]
                  + c4[:, 3:4] * walker[3:4, :])
            contrib = jax.lax.dot_general(
                ev.astype(jnp.bfloat16), swt, (((1,), (1,)), ((), ())),
                preferred_element_type=jnp.float32)
            walker = contrib + iso * ev
            walker = walker * jnp.exp(step.astype(jnp.float32)
                                      * (-_LOSS_DB / 10.0))
            norm = jnp.sqrt(jnp.sum(walker * walker))
            return walker / (norm + 1e-08)

        walker = jax.lax.fori_loop(0, _NSTEPS, _step, walker0)
        p0 = walker[0:1, :] ** 2 + walker[1:2, :] ** 2
        p1 = walker[2:3, :] ** 2 + walker[3:4, :] ** 2
        out_ref[b] = jnp.concatenate([p0, p1], axis=0)


def kernel(graph_adjacency, coin_operator, enc_W, enc_b, fe_W1, fe_b1,
           fe_W2, fe_b2):
    coin_c = coin_operator[..., 0] + 1j * coin_operator[..., 1]
    coin_c = coin_c / jnp.linalg.norm(coin_c)
    cr = jnp.real(coin_c).astype(jnp.float32)
    ci = jnp.imag(coin_c).astype(jnp.float32)
    c4 = jnp.stack([
        jnp.stack([cr[0, 0], -ci[0, 0], cr[0, 1], -ci[0, 1]]),
        jnp.stack([ci[0, 0], cr[0, 0], ci[0, 1], cr[0, 1]]),
        jnp.stack([cr[1, 0], -ci[1, 0], cr[1, 1], -ci[1, 1]]),
        jnp.stack([ci[1, 0], cr[1, 0], ci[1, 1], cr[1, 1]]),
    ])

    swt, iso = pl.pallas_call(
        _encoder_kernel,
        grid=(_GRID,),
        in_specs=[
            pl.BlockSpec((1, _TILE, _N), lambda t: (t // _NT, t % _NT, 0)),
            pl.BlockSpec((_N, _N), lambda t: (0, 0)),
        ],
        out_specs=[
            pl.BlockSpec((1, _N, _TILE), lambda t: (t // _NT, 0, t % _NT)),
            pl.BlockSpec((1, 1, _TILE), lambda t: (t // _NT, 0, t % _NT)),
        ],
        out_shape=[
            jax.ShapeDtypeStruct((_B, _N, _N), jnp.bfloat16),
            jax.ShapeDtypeStruct((_B, 1, _N), jnp.float32),
        ],
    )(graph_adjacency, enc_W)

    probs = pl.pallas_call(
        _walk_kernel,
        in_specs=[
            pl.BlockSpec((_B, _N, _N), lambda: (0, 0, 0)),
            pl.BlockSpec((_B, 1, _N), lambda: (0, 0, 0)),
            pl.BlockSpec((4, 4), lambda: (0, 0)),
        ],
        out_specs=pl.BlockSpec((_B, 2, _N), lambda: (0, 0, 0)),
        out_shape=jax.ShapeDtypeStruct((_B, 2, _N), jnp.float32),
    )(swt, iso, c4)

    flat_probs = jnp.transpose(probs, (0, 2, 1)).reshape(_B, -1)
    h = jax.nn.relu(flat_probs @ fe_W1.T + fe_b1)
    return h @ fe_W2.T + fe_b2


# speedup vs baseline: 1.6318x; 1.0196x over previous
import math

import jax
import jax.numpy as jnp
from jax.experimental import pallas as pl
from jax.experimental.pallas import tpu as pltpu

_N = 2048
_B = 2
_CD = 2
_TILE = 256
_NT = _N // _TILE
_GRID = _B * _NT
_NSTEPS = 8
_LOSS_DB = 0.1


def _encoder_kernel(adj_ref, encw_ref, swt_ref, iso_ref):
    logits_t = jax.lax.dot_general(
        encw_ref[...], adj_ref[0], (((1,), (1,)), ((), ())),
        preferred_element_type=jnp.float32)
    maskf = (jax.nn.sigmoid(logits_t) > 0.5).astype(jnp.float32)
    deg = jnp.sum(maskf, axis=0, keepdims=True)
    s = jnp.where(deg > 0, 1.0 / jnp.sqrt(jnp.maximum(deg, 1.0)), 0.0)
    swt_ref[0] = (maskf * s).astype(jnp.bfloat16)
    iso_ref[0] = (deg == 0.0).astype(jnp.float32)


def _walk_kernel(swt_ref, iso_ref, c4_ref, out_ref):
    c4 = c4_ref[...]
    swa, swb = swt_ref[0], swt_ref[1]
    isoa, isob = iso_ref[0], iso_ref[1]
    w0 = jnp.full((4, _N), 1.0 / math.sqrt(_N * _CD), dtype=jnp.float32)

    def _coin(walker):
        return (c4[:, 0:1] * walker[0:1, :]
                + c4[:, 1:2] * walker[1:2, :]
                + c4[:, 2:3] * walker[2:3, :]
                + c4[:, 3:4] * walker[3:4, :])

    def _step(step, carry):
        wa, wb = carry
        eva, evb = _coin(wa), _coin(wb)
        ca = jax.lax.dot_general(
            eva.astype(jnp.bfloat16), swa, (((1,), (1,)), ((), ())),
            preferred_element_type=jnp.float32)
        cb = jax.lax.dot_general(
            evb.astype(jnp.bfloat16), swb, (((1,), (1,)), ((), ())),
            preferred_element_type=jnp.float32)
        loss = jnp.exp(step.astype(jnp.float32) * (-_LOSS_DB / 10.0))
        wa = (ca + isoa * eva) * loss
        wb = (cb + isob * evb) * loss
        wa = wa / (jnp.sqrt(jnp.sum(wa * wa)) + 1e-08)
        wb = wb / (jnp.sqrt(jnp.sum(wb * wb)) + 1e-08)
        return wa, wb

    wa, wb = jax.lax.fori_loop(0, _NSTEPS, _step, (w0, w0))
    for b, w in ((0, wa), (1, wb)):
        p0 = w[0:1, :] ** 2 + w[1:2, :] ** 2
        p1 = w[2:3, :] ** 2 + w[3:4, :] ** 2
        out_ref[b] = jnp.concatenate([p0, p1], axis=0)


def kernel(graph_adjacency, coin_operator, enc_W, enc_b, fe_W1, fe_b1,
           fe_W2, fe_b2):
    coin_c = coin_operator[..., 0] + 1j * coin_operator[..., 1]
    coin_c = coin_c / jnp.linalg.norm(coin_c)
    cr = jnp.real(coin_c).astype(jnp.float32)
    ci = jnp.imag(coin_c).astype(jnp.float32)
    c4 = jnp.stack([
        jnp.stack([cr[0, 0], -ci[0, 0], cr[0, 1], -ci[0, 1]]),
        jnp.stack([ci[0, 0], cr[0, 0], ci[0, 1], cr[0, 1]]),
        jnp.stack([cr[1, 0], -ci[1, 0], cr[1, 1], -ci[1, 1]]),
        jnp.stack([ci[1, 0], cr[1, 0], ci[1, 1], cr[1, 1]]),
    ])

    swt, iso = pl.pallas_call(
        _encoder_kernel,
        grid=(_GRID,),
        in_specs=[
            pl.BlockSpec((1, _TILE, _N), lambda t: (t // _NT, t % _NT, 0)),
            pl.BlockSpec((_N, _N), lambda t: (0, 0)),
        ],
        out_specs=[
            pl.BlockSpec((1, _N, _TILE), lambda t: (t // _NT, 0, t % _NT)),
            pl.BlockSpec((1, 1, _TILE), lambda t: (t // _NT, 0, t % _NT)),
        ],
        out_shape=[
            jax.ShapeDtypeStruct((_B, _N, _N), jnp.bfloat16),
            jax.ShapeDtypeStruct((_B, 1, _N), jnp.float32),
        ],
    )(graph_adjacency, enc_W)

    probs = pl.pallas_call(
        _walk_kernel,
        in_specs=[
            pl.BlockSpec((_B, _N, _N), lambda: (0, 0, 0)),
            pl.BlockSpec((_B, 1, _N), lambda: (0, 0, 0)),
            pl.BlockSpec((4, 4), lambda: (0, 0)),
        ],
        out_specs=pl.BlockSpec((_B, 2, _N), lambda: (0, 0, 0)),
        out_shape=jax.ShapeDtypeStruct((_B, 2, _N), jnp.float32),
    )(swt, iso, c4)

    flat_probs = jnp.transpose(probs, (0, 2, 1)).reshape(_B, -1)
    h = jax.nn.relu(flat_probs @ fe_W1.T + fe_b1)
    return h @ fe_W2.T + fe_b2
